# baseline (speedup 1.0000x reference)
import functools

import jax
import jax.numpy as jnp
from jax import lax
from jax.experimental import pallas as pl
from jax.experimental.pallas import tpu as pltpu
from jax.experimental.pallas import tpu_sc as plsc

NC = 2
NS = 16
NW = NC * NS

G = 512
NG = 4
IC = G * NG
K = 2


def _make_kernel(n_total: int, d: int):
    n_per_w = n_total // NW
    n_iters = n_per_w // IC
    n_rounds = n_iters // K
    rows_per_w = n_per_w // G
    row_bytes = IC * d * 4

    mesh = plsc.VectorSubcoreMesh(core_axis_name="c", subcore_axis_name="s")

    @functools.partial(
        pl.kernel,
        mesh=mesh,
        compiler_params=pltpu.CompilerParams(use_tc_tiling_on_sc=False),
        out_type=jax.ShapeDtypeStruct((n_total, d), jnp.float32),
        scratch_types=[
            pltpu.VMEM((K, NG, G), jnp.int32),
            pltpu.VMEM((K, IC, d), jnp.float32),
        ]
        + [pltpu.SemaphoreType.DMA] * (3 * K),
    )
    def k(x_hbm, w_hbm, out_hbm, idx_v, rows_v, *sems):
        sem_g = sems[0:K]
        sem_s = sems[K:2 * K]
        sem_i = sems[2 * K:3 * K]
        wid = lax.axis_index("s") * NC + lax.axis_index("c")
        row_base = wid * rows_per_w
        out_base = wid * n_per_w

        def idx_src(i):
            row = row_base + jnp.minimum(i, n_iters - 1) * NG
            return x_hbm.at[pl.ds(row, NG), :]

        def fire_gathers(b, i):
            for j in range(NG):
                pltpu.async_copy(
                    w_hbm.at[idx_v.at[b, j]],
                    rows_v.at[b, pl.ds(j * G, G), :],
                    sem_g[b],
                )

        def fire_store(b, i):
            pltpu.async_copy(
                rows_v.at[b],
                out_hbm.at[pl.ds(out_base + i * IC, IC), :],
                sem_s[b],
            )

        def drain_gathers(b):
            pltpu.make_async_copy(
                out_hbm.at[pl.ds(0, IC), :], rows_v.at[b], sem_g[b]
            ).wait()

        def drain_store(b):
            pltpu.make_async_copy(
                rows_v.at[b], out_hbm.at[pl.ds(0, IC), :], sem_s[b]
            ).wait()

        def drain_idx(b):
            pltpu.make_async_copy(
                x_hbm.at[pl.ds(0, NG), :], idx_v.at[b], sem_i[b]
            ).wait()

        for b in range(K):
            pltpu.async_copy(idx_src(b), idx_v.at[b], sem_i[b])
        for b in range(K):
            drain_idx(b)
            fire_gathers(b, b)
        for b in range(K):
            drain_gathers(b)
            fire_store(b, b)
            pltpu.async_copy(idx_src(b + K), idx_v.at[b], sem_i[b])

        def body(r, carry):
            i0 = K * r
            for b in range(K):
                drain_store(b)
                drain_idx(b)
                fire_gathers(b, i0 + b)
            for b in range(K):
                drain_gathers(b)
                fire_store(b, i0 + b)
                pltpu.async_copy(idx_src(i0 + b + K), idx_v.at[b], sem_i[b])
            return carry

        lax.fori_loop(1, n_rounds, body, 0)

        for b in range(K):
            drain_store(b)
            drain_idx(b)

    return k


def kernel(x, W):
    b, t = x.shape
    n_total = b * t
    v, d = W.shape
    xf = x.reshape(n_total // G, G)
    one = lax.optimization_barrier(jnp.float32(1.0))
    wrm = (
        (W.T.reshape(d, v // 8, 8).transpose(1, 2, 0) * one)
        .reshape(v // 8, 8 * d)
        .reshape(v, d)
    )
    out = _make_kernel(n_total, d)(xf, wrm)
    return out.reshape(b, t, d)


# speedup vs baseline: 1.0703x; 1.0703x over previous
import functools

import jax
import jax.numpy as jnp
from jax import lax
from jax.experimental import pallas as pl
from jax.experimental.pallas import tpu as pltpu
from jax.experimental.pallas import tpu_sc as plsc

NC = 2
NS = 16
NW = NC * NS

G = 512
NG = 4
IC = G * NG
K = 2


BN = 8192


def _tc_transpose(wt, v: int, d: int):
    def body(i_ref, o_ref):
        t = i_ref[...].T.reshape(BN // 8, 8, d)
        o_ref[...] = jnp.concatenate([t[:, k, :] for k in range(8)], axis=1)

    return pl.pallas_call(
        body,
        grid=((v + BN - 1) // BN,),
        in_specs=[pl.BlockSpec((d, BN), lambda i: (0, i))],
        out_specs=pl.BlockSpec((BN // 8, 8 * d), lambda i: (i, 0)),
        out_shape=jax.ShapeDtypeStruct((v * d // 128, 128), jnp.float32),
    )(wt)


def _make_kernel(n_total: int, d: int):
    n_per_w = n_total // NW
    n_iters = n_per_w // IC
    n_rounds = n_iters // K
    rows_per_w = n_per_w // G
    row_bytes = IC * d * 4

    mesh = plsc.VectorSubcoreMesh(core_axis_name="c", subcore_axis_name="s")

    @functools.partial(
        pl.kernel,
        mesh=mesh,
        compiler_params=pltpu.CompilerParams(use_tc_tiling_on_sc=False),
        out_type=jax.ShapeDtypeStruct((n_total, d), jnp.float32),
        scratch_types=[
            pltpu.VMEM((K, NG, G), jnp.int32),
            pltpu.VMEM((K, IC, d), jnp.float32),
        ]
        + [pltpu.SemaphoreType.DMA] * (3 * K),
    )
    def k(x_hbm, w_hbm, out_hbm, idx_v, rows_v, *sems):
        sem_g = sems[0:K]
        sem_s = sems[K:2 * K]
        sem_i = sems[2 * K:3 * K]
        wid = lax.axis_index("s") * NC + lax.axis_index("c")
        row_base = wid * rows_per_w
        out_base = wid * n_per_w

        def idx_src(i):
            row = row_base + jnp.minimum(i, n_iters - 1) * NG
            return x_hbm.at[pl.ds(row, NG), :]

        def fire_gathers(b, i):
            for j in range(NG):
                pltpu.async_copy(
                    w_hbm.at[idx_v.at[b, j]],
                    rows_v.at[b, pl.ds(j * G, G), :],
                    sem_g[b],
                )

        def fire_store(b, i):
            pltpu.async_copy(
                rows_v.at[b],
                out_hbm.at[pl.ds(out_base + i * IC, IC), :],
                sem_s[b],
            )

        def drain_gathers(b):
            pltpu.make_async_copy(
                out_hbm.at[pl.ds(0, IC), :], rows_v.at[b], sem_g[b]
            ).wait()

        def drain_store(b):
            pltpu.make_async_copy(
                rows_v.at[b], out_hbm.at[pl.ds(0, IC), :], sem_s[b]
            ).wait()

        def drain_idx(b):
            pltpu.make_async_copy(
                x_hbm.at[pl.ds(0, NG), :], idx_v.at[b], sem_i[b]
            ).wait()

        for b in range(K):
            pltpu.async_copy(idx_src(b), idx_v.at[b], sem_i[b])
        for b in range(K):
            drain_idx(b)
            fire_gathers(b, b)
        for b in range(K):
            drain_gathers(b)
            fire_store(b, b)
            pltpu.async_copy(idx_src(b + K), idx_v.at[b], sem_i[b])

        def body(r, carry):
            i0 = K * r
            for b in range(K):
                drain_store(b)
                drain_idx(b)
                fire_gathers(b, i0 + b)
            for b in range(K):
                drain_gathers(b)
                fire_store(b, i0 + b)
                pltpu.async_copy(idx_src(i0 + b + K), idx_v.at[b], sem_i[b])
            return carry

        lax.fori_loop(1, n_rounds, body, 0)

        for b in range(K):
            drain_store(b)
            drain_idx(b)

    return k


def kernel(x, W):
    b, t = x.shape
    n_total = b * t
    v, d = W.shape
    xf = x.reshape(n_total // G, G)
    wrm = _tc_transpose(W.T, v, d).reshape(v, d)
    out = _make_kernel(n_total, d)(xf, wrm)
    return out.reshape(b, t, d)


# speedup vs baseline: 3.3313x; 3.1125x over previous
import functools

import jax
import jax.numpy as jnp
from jax import lax
from jax.experimental import pallas as pl
from jax.experimental.pallas import tpu as pltpu
from jax.experimental.pallas import tpu_sc as plsc

NC = 2
NS = 16
NW = NC * NS

G = 512
NG = 4
IC = G * NG
K = 2


BN = 8192


def _tc_transpose(wt, v: int, d: int):
    def body(i_ref, o_ref):
        t = i_ref[...].T.reshape(BN // 8, 8, d)
        o_ref[...] = jnp.concatenate([t[:, k, :] for k in range(8)], axis=1)

    return pl.pallas_call(
        body,
        grid=((v + BN - 1) // BN,),
        in_specs=[pl.BlockSpec((d, BN), lambda i: (0, i))],
        out_specs=pl.BlockSpec((BN // 8, 8 * d), lambda i: (i, 0)),
        out_shape=jax.ShapeDtypeStruct((v * d // 128, 128), jnp.float32),
    )(wt)


BB = 128


def _tc_outperm(i2, b_tot: int, t_tot: int, d: int):
    td = t_tot * d
    rows = td // 128

    def body(i_ref, o_ref):
        blk = i_ref[...]
        m = blk.reshape(BB, rows, 128).reshape(BB, td)
        o_ref[...] = m.T

    return pl.pallas_call(
        body,
        grid=(b_tot // BB,),
        in_specs=[pl.BlockSpec((rows * BB, 128), lambda i: (i, 0))],
        out_specs=pl.BlockSpec((td, BB), lambda i: (0, i)),
        out_shape=jax.ShapeDtypeStruct((td, b_tot), jnp.float32),
    )(i2)


def _make_kernel(n_total: int, d: int):
    n_per_w = n_total // NW
    n_iters = n_per_w // IC
    n_rounds = n_iters // K
    rows_per_w = n_per_w // G
    row_bytes = IC * d * 4

    mesh = plsc.VectorSubcoreMesh(core_axis_name="c", subcore_axis_name="s")

    @functools.partial(
        pl.kernel,
        mesh=mesh,
        compiler_params=pltpu.CompilerParams(use_tc_tiling_on_sc=False),
        out_type=jax.ShapeDtypeStruct((n_total, d), jnp.float32),
        scratch_types=[
            pltpu.VMEM((K, NG, G), jnp.int32),
            pltpu.VMEM((K, IC, d), jnp.float32),
        ]
        + [pltpu.SemaphoreType.DMA] * (3 * K),
    )
    def k(x_hbm, w_hbm, out_hbm, idx_v, rows_v, *sems):
        sem_g = sems[0:K]
        sem_s = sems[K:2 * K]
        sem_i = sems[2 * K:3 * K]
        wid = lax.axis_index("s") * NC + lax.axis_index("c")
        row_base = wid * rows_per_w
        out_base = wid * n_per_w

        def idx_src(i):
            row = row_base + jnp.minimum(i, n_iters - 1) * NG
            return x_hbm.at[pl.ds(row, NG), :]

        def fire_gathers(b, i):
            for j in range(NG):
                pltpu.async_copy(
                    w_hbm.at[idx_v.at[b, j]],
                    rows_v.at[b, pl.ds(j * G, G), :],
                    sem_g[b],
                )

        def fire_store(b, i):
            pltpu.async_copy(
                rows_v.at[b],
                out_hbm.at[pl.ds(out_base + i * IC, IC), :],
                sem_s[b],
            )

        def drain_gathers(b):
            pltpu.make_async_copy(
                out_hbm.at[pl.ds(0, IC), :], rows_v.at[b], sem_g[b]
            ).wait()

        def drain_store(b):
            pltpu.make_async_copy(
                rows_v.at[b], out_hbm.at[pl.ds(0, IC), :], sem_s[b]
            ).wait()

        def drain_idx(b):
            pltpu.make_async_copy(
                x_hbm.at[pl.ds(0, NG), :], idx_v.at[b], sem_i[b]
            ).wait()

        for b in range(K):
            pltpu.async_copy(idx_src(b), idx_v.at[b], sem_i[b])
        for b in range(K):
            drain_idx(b)
            fire_gathers(b, b)
        for b in range(K):
            drain_gathers(b)
            fire_store(b, b)
            pltpu.async_copy(idx_src(b + K), idx_v.at[b], sem_i[b])

        def body(r, carry):
            i0 = K * r
            for b in range(K):
                drain_store(b)
                drain_idx(b)
                fire_gathers(b, i0 + b)
            for b in range(K):
                drain_gathers(b)
                fire_store(b, i0 + b)
                pltpu.async_copy(idx_src(i0 + b + K), idx_v.at[b], sem_i[b])
            return carry

        lax.fori_loop(1, n_rounds, body, 0)

        for b in range(K):
            drain_store(b)
            drain_idx(b)

    return k


def kernel(x, W):
    b, t = x.shape
    n_total = b * t
    v, d = W.shape
    xf = x.reshape(n_total // G, G)
    wrm = _tc_transpose(W.T, v, d).reshape(v, d)
    out = _make_kernel(n_total, d)(xf, wrm)
    ot = _tc_outperm(out.reshape(n_total * d // 128, 128), b, t, d)
    return ot.reshape(t, d, b).transpose(2, 0, 1)


# speedup vs baseline: 3.3354x; 1.0012x over previous
import functools

import jax
import jax.numpy as jnp
from jax import lax
from jax.experimental import pallas as pl
from jax.experimental.pallas import tpu as pltpu
from jax.experimental.pallas import tpu_sc as plsc

NC = 2
NS = 16
NW = NC * NS

G = 512
NG = 5
IC = G * NG
K = 2


BN = 8192


def _tc_transpose(wt, v: int, d: int):
    def body(i_ref, o_ref):
        t = i_ref[...].T.reshape(BN // 8, 8, d)
        o_ref[...] = jnp.concatenate([t[:, k, :] for k in range(8)], axis=1)

    return pl.pallas_call(
        body,
        grid=((v + BN - 1) // BN,),
        in_specs=[pl.BlockSpec((d, BN), lambda i: (0, i))],
        out_specs=pl.BlockSpec((BN // 8, 8 * d), lambda i: (i, 0)),
        out_shape=jax.ShapeDtypeStruct((v * d // 128, 128), jnp.float32),
    )(wt)


BB = 128


def _tc_outperm(i2, b_tot: int, t_tot: int, d: int):
    td = t_tot * d
    rows = td // 128

    def body(i_ref, o_ref):
        blk = i_ref[...]
        m = blk.reshape(BB, rows, 128).reshape(BB, td)
        o_ref[...] = m.T

    return pl.pallas_call(
        body,
        grid=(b_tot // BB,),
        in_specs=[pl.BlockSpec((rows * BB, 128), lambda i: (i, 0))],
        out_specs=pl.BlockSpec((td, BB), lambda i: (0, i)),
        out_shape=jax.ShapeDtypeStruct((td, b_tot), jnp.float32),
    )(i2)


def _make_kernel(n_total: int, d: int):
    n_per_w = n_total // NW
    n_iters = n_per_w // IC
    n_rounds = n_iters // K
    rows_per_w = n_per_w // G
    row_bytes = IC * d * 4

    mesh = plsc.VectorSubcoreMesh(core_axis_name="c", subcore_axis_name="s")

    @functools.partial(
        pl.kernel,
        mesh=mesh,
        compiler_params=pltpu.CompilerParams(use_tc_tiling_on_sc=False),
        out_type=jax.ShapeDtypeStruct((n_total, d), jnp.float32),
        scratch_types=[
            pltpu.VMEM((K, NG, G), jnp.int32),
            pltpu.VMEM((K, IC, d), jnp.float32),
        ]
        + [pltpu.SemaphoreType.DMA] * (3 * K),
    )
    def k(x_hbm, w_hbm, out_hbm, idx_v, rows_v, *sems):
        sem_g = sems[0:K]
        sem_s = sems[K:2 * K]
        sem_i = sems[2 * K:3 * K]
        wid = lax.axis_index("s") * NC + lax.axis_index("c")
        row_base = wid * rows_per_w
        out_base = wid * n_per_w

        def idx_src(i):
            row = row_base + jnp.minimum(i, n_iters - 1) * NG
            return x_hbm.at[pl.ds(row, NG), :]

        def fire_gathers(b, i):
            for j in range(NG):
                pltpu.async_copy(
                    w_hbm.at[idx_v.at[b, j]],
                    rows_v.at[b, pl.ds(j * G, G), :],
                    sem_g[b],
                )

        def fire_store(b, i):
            pltpu.async_copy(
                rows_v.at[b],
                out_hbm.at[pl.ds(out_base + i * IC, IC), :],
                sem_s[b],
            )

        def drain_gathers(b):
            pltpu.make_async_copy(
                out_hbm.at[pl.ds(0, IC), :], rows_v.at[b], sem_g[b]
            ).wait()

        def drain_store(b):
            pltpu.make_async_copy(
                rows_v.at[b], out_hbm.at[pl.ds(0, IC), :], sem_s[b]
            ).wait()

        def drain_idx(b):
            pltpu.make_async_copy(
                x_hbm.at[pl.ds(0, NG), :], idx_v.at[b], sem_i[b]
            ).wait()

        for b in range(K):
            pltpu.async_copy(idx_src(b), idx_v.at[b], sem_i[b])
        for b in range(K):
            drain_idx(b)
            fire_gathers(b, b)
        for b in range(K):
            drain_gathers(b)
            fire_store(b, b)
            pltpu.async_copy(idx_src(b + K), idx_v.at[b], sem_i[b])

        def body(r, carry):
            i0 = K * r
            for b in range(K):
                drain_store(b)
                drain_idx(b)
                fire_gathers(b, i0 + b)
            for b in range(K):
                drain_gathers(b)
                fire_store(b, i0 + b)
                pltpu.async_copy(idx_src(i0 + b + K), idx_v.at[b], sem_i[b])
            return carry

        lax.fori_loop(1, n_rounds, body, 0)

        for b in range(K):
            drain_store(b)
            drain_idx(b)

    return k


def kernel(x, W):
    b, t = x.shape
    n_total = b * t
    v, d = W.shape
    xf = x.reshape(n_total // G, G)
    wrm = _tc_transpose(W.T, v, d).reshape(v, d)
    out = _make_kernel(n_total, d)(xf, wrm)
    ot = _tc_outperm(out.reshape(n_total * d // 128, 128), b, t, d)
    return ot.reshape(t, d, b).transpose(2, 0, 1)


# speedup vs baseline: 3.4947x; 1.0478x over previous
import functools

import jax
import jax.numpy as jnp
from jax import lax
from jax.experimental import pallas as pl
from jax.experimental.pallas import tpu as pltpu
from jax.experimental.pallas import tpu_sc as plsc

NC = 2
NS = 16
NW = NC * NS

G = 512
NG = 2
IC = G * NG
K = 2

BN = 8192
BB = 128


def _tc_transpose(wt, v: int, d: int):
    def body(i_ref, o_ref):
        t = i_ref[...].T.reshape(BN // 8, 8, d)
        o_ref[...] = jnp.concatenate([t[:, k, :] for k in range(8)], axis=1)

    return pl.pallas_call(
        body,
        grid=((v + BN - 1) // BN,),
        in_specs=[pl.BlockSpec((d, BN), lambda i: (0, i))],
        out_specs=pl.BlockSpec((BN // 8, 8 * d), lambda i: (i, 0)),
        out_shape=jax.ShapeDtypeStruct((v * d // 128, 128), jnp.float32),
    )(wt)


def _tc_outperm(i2, b_tot: int, b_half: int, t_tot: int, d: int,
                col_off: int, prev=None):
    td = t_tot * d
    rows = td // 128

    def body(*refs):
        i_ref, o_ref = refs[0], refs[-1]
        blk = i_ref[...]
        m = blk.reshape(BB, rows, 128).reshape(BB, td)
        o_ref[...] = m.T

    in_specs = [pl.BlockSpec((rows * BB, 128), lambda i: (i, 0))]
    operands = [i2]
    aliases = {}
    if prev is not None:
        in_specs.append(pl.BlockSpec(memory_space=pl.ANY))
        operands.append(prev)
        aliases = {1: 0}
    return pl.pallas_call(
        body,
        grid=(b_half // BB,),
        in_specs=in_specs,
        out_specs=pl.BlockSpec((td, BB), lambda i: (0, i + col_off)),
        out_shape=jax.ShapeDtypeStruct((td, b_tot), jnp.float32),
        input_output_aliases=aliases,
    )(*operands)


def _make_kernel(n_total: int, d: int, n_lo: int, n_count: int):
    n_per_w = n_count // NW
    n_iters = n_per_w // IC
    n_rounds = n_iters // K
    row_lo = n_lo // G
    rows_per_w = n_per_w // G

    mesh = plsc.VectorSubcoreMesh(core_axis_name="c", subcore_axis_name="s")

    @functools.partial(
        pl.kernel,
        mesh=mesh,
        compiler_params=pltpu.CompilerParams(use_tc_tiling_on_sc=False),
        out_type=jax.ShapeDtypeStruct((n_count, d), jnp.float32),
        scratch_types=[
            pltpu.VMEM((K, NG, G), jnp.int32),
            pltpu.VMEM((K, IC, d), jnp.float32),
        ]
        + [pltpu.SemaphoreType.DMA] * (3 * K),
    )
    def k(x_hbm, w_hbm, out_hbm, idx_v, rows_v, *sems):
        sem_g = sems[0:K]
        sem_s = sems[K:2 * K]
        sem_i = sems[2 * K:3 * K]
        wid = lax.axis_index("s") * NC + lax.axis_index("c")
        row_base = row_lo + wid * rows_per_w
        out_base = wid * n_per_w

        def idx_src(i):
            row = row_base + jnp.minimum(i, n_iters - 1) * NG
            return x_hbm.at[pl.ds(row, NG), :]

        def fire_gathers(b, i):
            for j in range(NG):
                pltpu.async_copy(
                    w_hbm.at[idx_v.at[b, j]],
                    rows_v.at[b, pl.ds(j * G, G), :],
                    sem_g[b],
                )

        def fire_store(b, i):
            pltpu.async_copy(
                rows_v.at[b],
                out_hbm.at[pl.ds(out_base + i * IC, IC), :],
                sem_s[b],
            )

        def drain_gathers(b):
            pltpu.make_async_copy(
                out_hbm.at[pl.ds(0, IC), :], rows_v.at[b], sem_g[b]
            ).wait()

        def drain_store(b):
            pltpu.make_async_copy(
                rows_v.at[b], out_hbm.at[pl.ds(0, IC), :], sem_s[b]
            ).wait()

        def drain_idx(b):
            pltpu.make_async_copy(
                x_hbm.at[pl.ds(0, NG), :], idx_v.at[b], sem_i[b]
            ).wait()

        for b in range(K):
            pltpu.async_copy(idx_src(b), idx_v.at[b], sem_i[b])
        for b in range(K):
            drain_idx(b)
            fire_gathers(b, b)
        for b in range(K):
            drain_gathers(b)
            fire_store(b, b)
            pltpu.async_copy(idx_src(b + K), idx_v.at[b], sem_i[b])

        def body(r, carry):
            i0 = K * r
            for b in range(K):
                drain_store(b)
                drain_idx(b)
                fire_gathers(b, i0 + b)
            for b in range(K):
                drain_gathers(b)
                fire_store(b, i0 + b)
                pltpu.async_copy(idx_src(i0 + b + K), idx_v.at[b], sem_i[b])
            return carry

        lax.fori_loop(1, n_rounds, body, 0)

        for b in range(K):
            drain_store(b)
            drain_idx(b)

    return k


def kernel(x, W):
    b, t = x.shape
    n_total = b * t
    v, d = W.shape
    n_half = n_total // 2
    b_half = b // 2
    xf = x.reshape(n_total // G, G)
    wrm = _tc_transpose(W.T, v, d).reshape(v, d)
    o1 = _make_kernel(n_total, d, 0, n_half)(xf, wrm)
    o2 = _make_kernel(n_total, d, n_half, n_half)(xf, wrm)
    ota = _tc_outperm(o1.reshape(n_half * d // 128, 128),
                      b, b_half, t, d, 0)
    ot = _tc_outperm(o2.reshape(n_half * d // 128, 128),
                     b, b_half, t, d, b_half // BB, prev=ota)
    return ot.reshape(t, d, b).transpose(2, 0, 1)
